# baseline (speedup 1.0000x reference)
import functools

import jax
import jax.numpy as jnp
from jax import lax
from jax.experimental import pallas as pl
from jax.experimental.pallas import tpu as pltpu
from jax.experimental.pallas import tpu_sc as plsc

_VOCAB = 100000
_EMB = 64
_B = 1024
_L = 50

_NC = 2
_NS = 16
_NW = _NC * _NS
_SAMP_PER_W = _B // _NW
_CHUNK = 100
_CHUNKS_PER_W = _SAMP_PER_W * _L // _CHUNK
_LANES = 16


def _pool_body(idx_hbm, table_hbm, out_hbm, idx_v, rows_v, out_v, sem):
    wid = lax.axis_index("s") * _NC + lax.axis_index("c")
    pltpu.sync_copy(idx_hbm.at[pl.ds(wid * _CHUNKS_PER_W, _CHUNKS_PER_W)], idx_v)
    copies = []
    for j in range(_CHUNKS_PER_W):
        copies.append(
            pltpu.async_copy(
                table_hbm.at[idx_v.at[j]],
                rows_v.at[pl.ds(j * _CHUNK, _CHUNK)],
                sem,
            )
        )
    for cp in copies:
        cp.wait()

    scale = jnp.float32(1.0 / _L)

    def sample_body(s, carry):
        base = s * _L
        acc = [jnp.zeros((_LANES,), jnp.float32) for _ in range(_EMB // _LANES)]
        for l in range(_L):
            r = base + l
            for k in range(_EMB // _LANES):
                acc[k] = acc[k] + rows_v[r, pl.ds(k * _LANES, _LANES)]
        for k in range(_EMB // _LANES):
            out_v[s, pl.ds(k * _LANES, _LANES)] = acc[k] * scale
        return carry

    lax.fori_loop(0, _SAMP_PER_W, sample_body, jnp.int32(0))
    pltpu.sync_copy(out_v, out_hbm.at[pl.ds(wid * _SAMP_PER_W, _SAMP_PER_W)])


_pool = functools.partial(
    pl.kernel,
    out_type=jax.ShapeDtypeStruct((_B, _EMB), jnp.float32),
    mesh=plsc.VectorSubcoreMesh(core_axis_name="c", subcore_axis_name="s"),
    scratch_types=[
        pltpu.VMEM((_CHUNKS_PER_W, _CHUNK), jnp.int32),
        pltpu.VMEM((_SAMP_PER_W * _L, _EMB), jnp.float32),
        pltpu.VMEM((_SAMP_PER_W, _EMB), jnp.float32),
        pltpu.SemaphoreType.DMA,
    ],
    compiler_params=pltpu.CompilerParams(use_tc_tiling_on_sc=False),
)(_pool_body)


_NBLK = 2048
_GRID_N = (_VOCAB + _NBLK - 1) // _NBLK


def _proj_body(x_ref, w_ref, b_ref, o_ref):
    o_ref[...] = (
        jnp.dot(x_ref[...], w_ref[...], preferred_element_type=jnp.float32)
        + b_ref[...]
    )


def _project(x, W, b2):
    return pl.pallas_call(
        _proj_body,
        grid=(_GRID_N,),
        in_specs=[
            pl.BlockSpec((_B, _EMB), lambda i: (0, 0)),
            pl.BlockSpec((_EMB, _NBLK), lambda i: (0, i)),
            pl.BlockSpec((1, _NBLK), lambda i: (0, i)),
        ],
        out_specs=pl.BlockSpec((_B, _NBLK), lambda i: (0, i)),
        out_shape=jax.ShapeDtypeStruct((_B, _VOCAB), jnp.float32),
        compiler_params=pltpu.CompilerParams(
            dimension_semantics=("arbitrary",),
        ),
    )(x, W, b2)


def kernel(inputs, emb_table, W, b):
    idx2d = inputs.reshape(_B * _L // _CHUNK, _CHUNK)
    x = _pool(idx2d, emb_table)
    return _project(x, W, b.reshape(1, _VOCAB))
